# baseline (speedup 1.0000x reference)
import functools

import jax
import jax.numpy as jnp
from jax import lax
from jax.experimental import pallas as pl
from jax.experimental.pallas import tpu as pltpu
from jax.experimental.pallas import tpu_sc as plsc

_B, _T, _C, _P, _R = 2048, 24, 4, 3, 17
_NC, _NS, _L = 2, 16, 16
_NW = _NC * _NS
_BW = _B // _NW
_NCH = _BW // _L
_RP = 24
_NTB = _P * _P + _P
_SH = 7

_C0 = 2.2117031200252768e-05 + 1e-4
_C1 = 0.9990104466294587
_C2 = -0.4891568472023044
_C3 = 0.28330432451740856
_C4 = -0.13011941539126315
_C5 = 0.03010262501167511


def _alpha(y):
    t = jnp.exp(-jnp.abs(y))
    t2 = t * t
    q01 = jnp.float32(_C1) * t + jnp.float32(_C0)
    q23 = jnp.float32(_C3) * t + jnp.float32(_C2)
    q45 = jnp.float32(_C5) * t + jnp.float32(_C4)
    p = (q45 * t2 + q23) * t2 + q01
    return jnp.maximum(y, jnp.float32(0.0)) + p


def _sc_body(ns_hbm, pk_hbm, tbl_hbm, out_hbm, ns_v, pk_v, sem, *tbl):
    wid = lax.axis_index("s") * _NC + lax.axis_index("c")
    col = wid * _BW
    copies = [
        pltpu.make_async_copy(ns_hbm.at[:, pl.ds(col, _BW)], ns_v, sem),
        pltpu.make_async_copy(pk_hbm.at[:, pl.ds(col, _BW)], pk_v, sem),
    ] + [
        pltpu.make_async_copy(tbl_hbm.at[pl.ds(k * _RP, _RP)], tbl[k], sem)
        for k in range(_NTB)
    ]
    for c in copies:
        c.start()
    for c in copies:
        c.wait()

    iota = lax.broadcasted_iota(jnp.int32, (_L,), 0)
    cols = [iota + ch * _L for ch in range(_NCH)]

    def step(i, carry):
        i3 = i * _P
        par = tuple(ns_v[i3 + q, pl.ds(ch * _L, _L)]
                    for ch in range(_NCH) for q in range(_P))

        def jstep(j, par):
            r = i * _C + j
            out = []
            for ch in range(_NCH):
                pk = pk_v[r, pl.ds(ch * _L, _L)]
                ca = pk & ((1 << _SH) - 1)
                rl = lax.shift_right_logical(pk, _SH)
                p0, p1, p2 = par[ch * _P], par[ch * _P + 1], par[ch * _P + 2]
                self_ref = ca == i3
                c0 = jnp.where(self_ref, p0, plsc.load_gather(ns_v, [ca, cols[ch]]))
                c1 = jnp.where(self_ref, p1, plsc.load_gather(ns_v, [ca + 1, cols[ch]]))
                c2 = jnp.where(self_ref, p2, plsc.load_gather(ns_v, [ca + 2, cols[ch]]))
                m = [plsc.load_gather(tbl[k], [rl]) for k in range(9)]
                b = [plsc.load_gather(tbl[9 + q], [rl]) for q in range(_P)]
                a0 = _alpha(p0 * (m[0] * c0 + m[1] * c1 + m[2] * c2) + b[0])
                a1 = _alpha(p1 * (m[3] * c0 + m[4] * c1 + m[5] * c2) + b[1])
                a2 = _alpha(p2 * (m[6] * c0 + m[7] * c1 + m[8] * c2) + b[2])
                rinv = jnp.float32(1.0) / (a0 + a1 + a2)
                keep = rl != 0
                out += [jnp.where(keep, a0 * rinv, p0),
                        jnp.where(keep, a1 * rinv, p1),
                        jnp.where(keep, a2 * rinv, p2)]
            return tuple(out)

        par = lax.fori_loop(0, _C, jstep, par)
        for ch in range(_NCH):
            for q in range(_P):
                ns_v[i3 + q, pl.ds(ch * _L, _L)] = par[ch * _P + q]
        return carry

    lax.fori_loop(0, _T, step, 0)
    pltpu.sync_copy(ns_v.at[pl.ds((_T - 1) * _P, _P), :],
                    out_hbm.at[:, pl.ds(col, _BW)])


@functools.partial(
    pl.kernel,
    out_type=jax.ShapeDtypeStruct((_P, _B), jnp.float32),
    mesh=plsc.VectorSubcoreMesh(core_axis_name="c", subcore_axis_name="s",
                                num_cores=_NC, num_subcores=_NS),
    compiler_params=pltpu.CompilerParams(needs_layout_passes=False,
                                         use_tc_tiling_on_sc=False,
                                         skip_device_barrier=True),
    scratch_types=[
        pltpu.VMEM((_T * _P, _BW), jnp.float32),
        pltpu.VMEM((_T * _C, _BW), jnp.int32),
        pltpu.SemaphoreType.DMA,
    ] + [pltpu.VMEM((_RP,), jnp.float32) for _ in range(_NTB)],
)
def _sc_kernel(*refs):
    _sc_body(*refs)


def kernel(node_scores, children, rels, labels, M, beta, scale):
    del labels
    ns_t = (node_scores.astype(jnp.float32)
            .transpose(1, 2, 0).reshape(_T * _P, _B))
    pk = (children.astype(jnp.int32) * _P
          + (rels.astype(jnp.int32) << _SH))
    pk_t = pk.transpose(1, 2, 0).reshape(_T * _C, _B)
    mt = (M.astype(jnp.float32) * scale).reshape(_R, _P * _P).transpose(1, 0)
    bt = beta.astype(jnp.float32).transpose(1, 0)
    tbl = jnp.pad(jnp.concatenate([mt, bt], axis=0), ((0, 0), (0, _RP - _R)))
    out = _sc_kernel(ns_t, pk_t, tbl.reshape(_NTB * _RP))
    return out.transpose(1, 0)


# speedup vs baseline: 1.0160x; 1.0160x over previous
import functools

import jax
import jax.numpy as jnp
from jax import lax
from jax.experimental import pallas as pl
from jax.experimental.pallas import tpu as pltpu
from jax.experimental.pallas import tpu_sc as plsc

_B, _T, _C, _P, _R = 2048, 24, 4, 3, 17
_NC, _NS, _L = 2, 16, 16
_NW = _NC * _NS
_BW = _B // _NW
_NCH = _BW // _L
_RP = 24
_NTB = _P * _P + _P
_SH = 7

_C0 = 0.00014151217537855532 + 1e-4
_C1 = 0.9954273382579939
_C2 = -0.4640725804471406
_C3 = 0.21641043832783918
_C4 = -0.054862852862074235


def _alpha(y):
    t = jnp.exp(-jnp.abs(y))
    t2 = t * t
    q01 = jnp.float32(_C1) * t + jnp.float32(_C0)
    q23 = jnp.float32(_C3) * t + jnp.float32(_C2)
    p = (jnp.float32(_C4) * t2 + q23) * t2 + q01
    return jnp.maximum(y, jnp.float32(0.0)) + p


def _sc_body(ns_hbm, pk_hbm, tbl_hbm, out_hbm, ns_v, pk_v, sem, *tbl):
    wid = lax.axis_index("s") * _NC + lax.axis_index("c")
    col = wid * _BW
    copies = [
        pltpu.make_async_copy(ns_hbm.at[:, pl.ds(col, _BW)], ns_v, sem),
        pltpu.make_async_copy(pk_hbm.at[:, pl.ds(col, _BW)], pk_v, sem),
    ] + [
        pltpu.make_async_copy(tbl_hbm.at[pl.ds(k * _RP, _RP)], tbl[k], sem)
        for k in range(_NTB)
    ]
    for c in copies:
        c.start()
    for c in copies:
        c.wait()

    iota = lax.broadcasted_iota(jnp.int32, (_L,), 0)
    cols = [iota + ch * _L for ch in range(_NCH)]

    def step(i, carry):
        i3 = i * _P
        par = tuple(ns_v[i3 + q, pl.ds(ch * _L, _L)]
                    for ch in range(_NCH) for q in range(_P))

        def jstep(j, par):
            r = i * _C + j
            out = []
            for ch in range(_NCH):
                pk = pk_v[r, pl.ds(ch * _L, _L)]
                ca = pk & ((1 << _SH) - 1)
                rl = lax.shift_right_logical(pk, _SH)
                p0, p1, p2 = par[ch * _P], par[ch * _P + 1], par[ch * _P + 2]
                self_ref = ca == i3
                c0 = jnp.where(self_ref, p0, plsc.load_gather(ns_v, [ca, cols[ch]]))
                c1 = jnp.where(self_ref, p1, plsc.load_gather(ns_v, [ca + 1, cols[ch]]))
                c2 = jnp.where(self_ref, p2, plsc.load_gather(ns_v, [ca + 2, cols[ch]]))
                m = [plsc.load_gather(tbl[k], [rl]) for k in range(9)]
                b = [plsc.load_gather(tbl[9 + q], [rl]) for q in range(_P)]
                a0 = _alpha(p0 * (m[0] * c0 + m[1] * c1 + m[2] * c2) + b[0])
                a1 = _alpha(p1 * (m[3] * c0 + m[4] * c1 + m[5] * c2) + b[1])
                a2 = _alpha(p2 * (m[6] * c0 + m[7] * c1 + m[8] * c2) + b[2])
                rinv = jnp.float32(1.0) / (a0 + a1 + a2)
                keep = rl != 0
                out += [jnp.where(keep, a0 * rinv, p0),
                        jnp.where(keep, a1 * rinv, p1),
                        jnp.where(keep, a2 * rinv, p2)]
            return tuple(out)

        par = lax.fori_loop(0, _C, jstep, par)
        for ch in range(_NCH):
            for q in range(_P):
                ns_v[i3 + q, pl.ds(ch * _L, _L)] = par[ch * _P + q]
        return carry

    lax.fori_loop(0, _T, step, 0)
    pltpu.sync_copy(ns_v.at[pl.ds((_T - 1) * _P, _P), :],
                    out_hbm.at[:, pl.ds(col, _BW)])


@functools.partial(
    pl.kernel,
    out_type=jax.ShapeDtypeStruct((_P, _B), jnp.float32),
    mesh=plsc.VectorSubcoreMesh(core_axis_name="c", subcore_axis_name="s",
                                num_cores=_NC, num_subcores=_NS),
    compiler_params=pltpu.CompilerParams(needs_layout_passes=False,
                                         use_tc_tiling_on_sc=False),
    scratch_types=[
        pltpu.VMEM((_T * _P, _BW), jnp.float32),
        pltpu.VMEM((_T * _C, _BW), jnp.int32),
        pltpu.SemaphoreType.DMA,
    ] + [pltpu.VMEM((_RP,), jnp.float32) for _ in range(_NTB)],
)
def _sc_kernel(*refs):
    _sc_body(*refs)


def kernel(node_scores, children, rels, labels, M, beta, scale):
    del labels
    ns_t = (node_scores.astype(jnp.float32)
            .transpose(1, 2, 0).reshape(_T * _P, _B))
    pk = (children.astype(jnp.int32) * _P
          + (rels.astype(jnp.int32) << _SH))
    pk_t = pk.transpose(1, 2, 0).reshape(_T * _C, _B)
    mt = (M.astype(jnp.float32) * scale).reshape(_R, _P * _P).transpose(1, 0)
    bt = beta.astype(jnp.float32).transpose(1, 0)
    tbl = jnp.pad(jnp.concatenate([mt, bt], axis=0), ((0, 0), (0, _RP - _R)))
    out = _sc_kernel(ns_t, pk_t, tbl.reshape(_NTB * _RP))
    return out.transpose(1, 0)
